# baseline (speedup 1.0000x reference)
import jax
import jax.numpy as jnp
from jax import lax
from jax.experimental import pallas as pl
from jax.experimental.pallas import tpu as pltpu
from jax.experimental.pallas import tpu_sc as plsc

NW = 32
CHUNK = 48
LANES = 16
NPL = 3


def _make_sc_sampler(N, H, W, C):
    mesh = plsc.VectorSubcoreMesh(core_axis_name="c", subcore_axis_name="s")
    nfull, tail = divmod(N, CHUNK)
    ncb, rem = divmod(nfull, NW)
    ppw = (ncb + 1) * CHUNK
    start_last = CHUNK * (ncb * (NW - 1) + min(NW - 1, rem))
    p_pad = start_last + ppw
    inv_scale = jnp.float32(1.0 / (1 + 0.1 + 10e-4))
    plane_xy = ((0, 2), (0, 1), (1, 2))

    def body(p0_h, p1_h, p2_h, tab_h, out_h, *sc):
        pb = sc[0:3]
        idxb = [sc[3 + 2 * p:5 + 2 * p] for p in range(NPL)]
        wbuf = [sc[9 + 2 * p:11 + 2 * p] for p in range(NPL)]
        rows = sc[15:18]
        outv = sc[18:21]
        gsem = sc[21:24]
        osem = sc[24:27]

        wid = lax.axis_index("s") * 2 + lax.axis_index("c")
        base0 = CHUNK * (ncb * wid + jnp.minimum(wid, rem))
        nfull_w = ncb + (wid < rem).astype(jnp.int32)
        has_tail = (wid == NW - 1) if tail else False
        for k, ph in enumerate((p0_h, p1_h, p2_h)):
            pltpu.sync_copy(ph.at[pl.ds(base0, ppw)], pb[k])

        def norm_to_coord(t, extent):
            t = t * inv_scale
            t = t + 0.5
            t = jnp.where(t >= 1.0, jnp.float32(1 - 10e-4), t)
            t = jnp.where(t < 0.0, jnp.float32(0.0), t)
            g = 2.0 * t - 1.0
            f = (g + 1.0) * 0.5 * (extent - 1)
            f = jnp.minimum(jnp.maximum(f, 0.0), jnp.float32(extent - 1))
            return f

        def compute_and_fire(plane, ch2):
            xb = pb[plane_xy[plane][0]]
            yb = pb[plane_xy[plane][1]]
            off = plane * (H * W)
            cb = ch2 * CHUNK
            for g in range(CHUNK // LANES):
                fx = norm_to_coord(xb[pl.ds(cb + g * LANES, LANES)], W)
                fy = norm_to_coord(yb[pl.ds(cb + g * LANES, LANES)], H)
                x0 = fx.astype(jnp.int32)
                y0 = fy.astype(jnp.int32)
                wx = fx - x0.astype(jnp.float32)
                wy = fy - y0.astype(jnp.float32)
                idx = off + y0 * W + x0
                s = pl.ds(g * LANES, LANES)
                idxb[plane][0][s] = idx
                idxb[plane][1][s] = idx + W
                wbuf[plane][0][s] = wx
                wbuf[plane][1][s] = wy
            for k in range(2):
                pltpu.async_copy(tab_h.at[idxb[plane][k]],
                                 rows[plane].at[pl.ds(k * CHUNK, CHUNK)],
                                 gsem[plane])

        def out_slice(plane, ch, npts=CHUNK):
            base = base0 + ch * CHUNK
            return out_h.at[pl.ds(base, npts), pl.ds(plane * C, C)]

        def wait_gathers(plane):
            pltpu.make_async_copy(tab_h.at[pl.ds(0, 2 * CHUNK)], rows[plane],
                                  gsem[plane]).wait()

        def do_combine(plane, npts):
            rb = rows[plane]

            @plsc.parallel_loop(0, npts // LANES, step=1)
            def comb(g2):
                gs = pl.ds(g2 * LANES, LANES)
                wxv = wbuf[plane][0][gs]
                wyv = wbuf[plane][1][gs]
                for i2 in range(LANES):
                    i = g2 * LANES + i2
                    wx = wxv[i2]
                    wy = wyv[i2]
                    for j in range(C // LANES):
                        ls = pl.ds(j * LANES, LANES)
                        rs = pl.ds(C + j * LANES, LANES)
                        t0 = rb[i, ls]
                        t1 = rb[i + CHUNK, ls]
                        h0 = t0 + wx * (rb[i, rs] - t0)
                        h1 = t1 + wx * (rb[i + CHUNK, rs] - t1)
                        outv[plane][i, ls] = h0 + wy * (h1 - h0)

        def combine(plane, ch):
            wait_gathers(plane)

            @pl.when(ch > 0)
            def _wait_prev_store():
                pltpu.make_async_copy(outv[plane], out_slice(plane, ch - 1),
                                      osem[plane]).wait()

            do_combine(plane, CHUNK)
            pltpu.async_copy(outv[plane], out_slice(plane, ch), osem[plane])

        for plane in range(NPL):
            compute_and_fire(plane, 0)

        def chunk_body(ch, carry):
            for plane in range(NPL):
                combine(plane, ch)

                @pl.when(ch + 1 < nfull_w)
                def _fire_next(plane=plane):
                    compute_and_fire(plane, ch + 1)
            return carry

        lax.fori_loop(0, nfull_w, chunk_body, 0)

        if tail:
            @pl.when(has_tail)
            def _tail():
                for plane in range(NPL):
                    compute_and_fire(plane, nfull_w)
                for plane in range(NPL):
                    wait_gathers(plane)
                    pltpu.make_async_copy(
                        outv[plane], out_slice(plane, nfull_w - 1),
                        osem[plane]).wait()
                    do_combine(plane, tail)
                    pltpu.async_copy(outv[plane].at[pl.ds(0, tail)],
                                     out_slice(plane, nfull_w, tail),
                                     osem[plane])
                for plane in range(NPL):
                    pltpu.make_async_copy(outv[plane].at[pl.ds(0, tail)],
                                          out_slice(plane, nfull_w, tail),
                                          osem[plane]).wait()

            @pl.when(jnp.logical_not(has_tail))
            def _no_tail():
                for plane in range(NPL):
                    pltpu.make_async_copy(outv[plane],
                                          out_slice(plane, nfull_w - 1),
                                          osem[plane]).wait()
        else:
            for plane in range(NPL):
                pltpu.make_async_copy(outv[plane],
                                      out_slice(plane, nfull_w - 1),
                                      osem[plane]).wait()

    scratch = (
        [pltpu.VMEM((ppw,), jnp.float32) for _ in range(3)]
        + [pltpu.VMEM((CHUNK,), jnp.int32) for _ in range(2 * NPL)]
        + [pltpu.VMEM((CHUNK,), jnp.float32) for _ in range(2 * NPL)]
        + [pltpu.VMEM((2 * CHUNK, 2 * C), jnp.float32) for _ in range(NPL)]
        + [pltpu.VMEM((CHUNK, C), jnp.float32) for _ in range(NPL)]
        + [pltpu.SemaphoreType.DMA for _ in range(2 * NPL)]
    )
    return pl.kernel(
        body,
        out_type=jax.ShapeDtypeStruct((N, NPL * C), jnp.float32),
        mesh=mesh,
        scratch_types=scratch,
    ), p_pad


@jax.jit
def kernel(p, c_xz, c_xy, c_yz):
    B, N, _ = p.shape
    _, C, Hh, Ww = c_xz.shape
    tabs = [c[0].reshape(C, Hh * Ww).T for c in (c_xz, c_xy, c_yz)]
    tab = jnp.concatenate(tabs, axis=0)
    tab = jnp.concatenate([tab, jnp.roll(tab, -1, axis=0)], axis=1)
    sampler, p_pad = _make_sc_sampler(N, Hh, Ww, C)
    pt = jnp.pad(p[0].T, ((0, 0), (0, p_pad - N)))
    out = sampler(pt[0], pt[1], pt[2], tab)
    return out[None]


# speedup vs baseline: 5.0730x; 5.0730x over previous
import jax
import jax.numpy as jnp
from jax import lax
from jax.experimental import pallas as pl
from jax.experimental.pallas import tpu as pltpu
from jax.experimental.pallas import tpu_sc as plsc

NW = 32
CHUNK = 48
LANES = 16
NPL = 3


def _make_sc_sampler(N, H, W, C):
    mesh = plsc.VectorSubcoreMesh(core_axis_name="c", subcore_axis_name="s")
    nfull, tail = divmod(N, CHUNK)
    ncb, rem = divmod(nfull, NW)
    ppw = (ncb + 1) * CHUNK
    start_last = CHUNK * (ncb * (NW - 1) + min(NW - 1, rem))
    p_pad = start_last + ppw
    inv_scale = jnp.float32(1.0 / (1 + 0.1 + 10e-4))
    plane_xy = ((0, 2), (0, 1), (1, 2))

    def body(p0_h, p1_h, p2_h, tab_h, out_h, *sc):
        pb = sc[0:3]
        idxb = [sc[3 + 4 * p:7 + 4 * p] for p in range(NPL)]
        wbuf = [sc[15 + 2 * p:17 + 2 * p] for p in range(NPL)]
        rows = sc[21:24]
        outv = sc[24:27]
        gsem = sc[27:30]
        osem = sc[30:33]

        wid = lax.axis_index("s") * 2 + lax.axis_index("c")
        base0 = CHUNK * (ncb * wid + jnp.minimum(wid, rem))
        nfull_w = ncb + (wid < rem).astype(jnp.int32)
        has_tail = (wid == NW - 1) if tail else False
        for k, ph in enumerate((p0_h, p1_h, p2_h)):
            pltpu.sync_copy(ph.at[pl.ds(base0, ppw)], pb[k])

        def norm_to_coord(t, extent):
            t = t * inv_scale
            t = t + 0.5
            t = jnp.where(t >= 1.0, jnp.float32(1 - 10e-4), t)
            t = jnp.where(t < 0.0, jnp.float32(0.0), t)
            g = 2.0 * t - 1.0
            f = (g + 1.0) * 0.5 * (extent - 1)
            f = jnp.minimum(jnp.maximum(f, 0.0), jnp.float32(extent - 1))
            return f

        def compute_and_fire(plane, ch2):
            xb = pb[plane_xy[plane][0]]
            yb = pb[plane_xy[plane][1]]
            off = plane * (H * W)
            cb = ch2 * CHUNK
            for g in range(CHUNK // LANES):
                fx = norm_to_coord(xb[pl.ds(cb + g * LANES, LANES)], W)
                fy = norm_to_coord(yb[pl.ds(cb + g * LANES, LANES)], H)
                x0 = fx.astype(jnp.int32)
                y0 = fy.astype(jnp.int32)
                wx = fx - x0.astype(jnp.float32)
                wy = fy - y0.astype(jnp.float32)
                idx = off + y0 * W + x0
                s = pl.ds(g * LANES, LANES)
                idxb[plane][0][s] = idx
                idxb[plane][1][s] = idx + 1
                idxb[plane][2][s] = idx + W
                idxb[plane][3][s] = idx + W + 1
                wbuf[plane][0][s] = wx
                wbuf[plane][1][s] = wy
            for k in range(4):
                pltpu.async_copy(tab_h.at[idxb[plane][k]],
                                 rows[plane].at[pl.ds(k * CHUNK, CHUNK)],
                                 gsem[plane])

        def out_slice(plane, ch, npts=CHUNK):
            base = base0 + ch * CHUNK
            return out_h.at[pl.ds(base, npts), pl.ds(plane * C, C)]

        def wait_gathers(plane):
            pltpu.make_async_copy(tab_h.at[pl.ds(0, 4 * CHUNK)], rows[plane],
                                  gsem[plane]).wait()

        def do_combine(plane, npts):
            rb = rows[plane]

            @plsc.parallel_loop(0, npts // LANES, step=1)
            def comb(g2):
                gs = pl.ds(g2 * LANES, LANES)
                wxv = wbuf[plane][0][gs]
                wyv = wbuf[plane][1][gs]
                for i2 in range(LANES):
                    i = g2 * LANES + i2
                    wx = wxv[i2]
                    wy = wyv[i2]
                    for j in range(C // LANES):
                        ls = pl.ds(j * LANES, LANES)
                        t0 = rb[i, ls]
                        t1 = rb[i + 2 * CHUNK, ls]
                        h0 = t0 + wx * (rb[i + CHUNK, ls] - t0)
                        h1 = t1 + wx * (rb[i + 3 * CHUNK, ls] - t1)
                        outv[plane][i, ls] = h0 + wy * (h1 - h0)

        def combine(plane, ch):
            wait_gathers(plane)

            @pl.when(ch > 0)
            def _wait_prev_store():
                pltpu.make_async_copy(outv[plane], out_slice(plane, ch - 1),
                                      osem[plane]).wait()

            do_combine(plane, CHUNK)
            pltpu.async_copy(outv[plane], out_slice(plane, ch), osem[plane])

        for plane in range(NPL):
            compute_and_fire(plane, 0)

        def chunk_body(ch, carry):
            for plane in range(NPL):
                combine(plane, ch)

                @pl.when(ch + 1 < nfull_w)
                def _fire_next(plane=plane):
                    compute_and_fire(plane, ch + 1)
            return carry

        lax.fori_loop(0, nfull_w, chunk_body, 0)

        if tail:
            @pl.when(has_tail)
            def _tail():
                for plane in range(NPL):
                    compute_and_fire(plane, nfull_w)
                for plane in range(NPL):
                    wait_gathers(plane)
                    pltpu.make_async_copy(
                        outv[plane], out_slice(plane, nfull_w - 1),
                        osem[plane]).wait()
                    do_combine(plane, tail)
                    pltpu.async_copy(outv[plane].at[pl.ds(0, tail)],
                                     out_slice(plane, nfull_w, tail),
                                     osem[plane])
                for plane in range(NPL):
                    pltpu.make_async_copy(outv[plane].at[pl.ds(0, tail)],
                                          out_slice(plane, nfull_w, tail),
                                          osem[plane]).wait()

            @pl.when(jnp.logical_not(has_tail))
            def _no_tail():
                for plane in range(NPL):
                    pltpu.make_async_copy(outv[plane],
                                          out_slice(plane, nfull_w - 1),
                                          osem[plane]).wait()
        else:
            for plane in range(NPL):
                pltpu.make_async_copy(outv[plane],
                                      out_slice(plane, nfull_w - 1),
                                      osem[plane]).wait()

    scratch = (
        [pltpu.VMEM((ppw,), jnp.float32) for _ in range(3)]
        + [pltpu.VMEM((CHUNK,), jnp.int32) for _ in range(4 * NPL)]
        + [pltpu.VMEM((CHUNK,), jnp.float32) for _ in range(2 * NPL)]
        + [pltpu.VMEM((4 * CHUNK, C), jnp.float32) for _ in range(NPL)]
        + [pltpu.VMEM((CHUNK, C), jnp.float32) for _ in range(NPL)]
        + [pltpu.SemaphoreType.DMA for _ in range(2 * NPL)]
    )
    return pl.kernel(
        body,
        out_type=jax.ShapeDtypeStruct((N, NPL * C), jnp.float32),
        mesh=mesh,
        scratch_types=scratch,
    ), p_pad


@jax.jit
def kernel(p, c_xz, c_xy, c_yz):
    B, N, _ = p.shape
    _, C, Hh, Ww = c_xz.shape
    tabs = [c[0].reshape(C, Hh * Ww).T for c in (c_xz, c_xy, c_yz)]
    tab = jnp.concatenate(tabs, axis=0)
    sampler, p_pad = _make_sc_sampler(N, Hh, Ww, C)
    pt = jnp.pad(p[0].T, ((0, 0), (0, p_pad - N)))
    out = sampler(pt[0], pt[1], pt[2], tab)
    return out[None]


# speedup vs baseline: 5.1456x; 1.0143x over previous
import jax
import jax.numpy as jnp
from jax import lax
from jax.experimental import pallas as pl
from jax.experimental.pallas import tpu as pltpu
from jax.experimental.pallas import tpu_sc as plsc

NW = 32
CHUNK = 48
LANES = 16
NPL = 3


def _tc_transpose(c2):
    C, HW = c2.shape
    BLK = 2048

    def tbody(in_ref, out_ref):
        out_ref[...] = in_ref[...].T

    return pl.pallas_call(
        tbody,
        grid=(HW // BLK,),
        in_specs=[pl.BlockSpec((C, BLK), lambda b: (0, b))],
        out_specs=pl.BlockSpec((BLK, C), lambda b: (b, 0)),
        out_shape=jax.ShapeDtypeStruct((HW, C), jnp.float32),
    )(c2)


def _make_sc_sampler(N, H, W, C):
    mesh = plsc.VectorSubcoreMesh(core_axis_name="c", subcore_axis_name="s")
    nfull, tail = divmod(N, CHUNK)
    ncb, rem = divmod(nfull, NW)
    ppw = (ncb + 1) * CHUNK
    start_last = CHUNK * (ncb * (NW - 1) + min(NW - 1, rem))
    p_pad = start_last + ppw
    inv_scale = jnp.float32(1.0 / (1 + 0.1 + 10e-4))
    plane_xy = ((0, 2), (0, 1), (1, 2))

    def body(p0_h, p1_h, p2_h, tab0_h, tab1_h, tab2_h, out_h, *sc):
        tabs_h = (tab0_h, tab1_h, tab2_h)
        pb = sc[0:3]
        idxb = [sc[3 + 4 * p:7 + 4 * p] for p in range(NPL)]
        wbuf = [sc[15 + 2 * p:17 + 2 * p] for p in range(NPL)]
        rows = sc[21:24]
        outv = sc[24:27]
        gsem = sc[27:30]
        osem = sc[30:33]

        wid = lax.axis_index("s") * 2 + lax.axis_index("c")
        base0 = CHUNK * (ncb * wid + jnp.minimum(wid, rem))
        nfull_w = ncb + (wid < rem).astype(jnp.int32)
        has_tail = (wid == NW - 1) if tail else False
        for k, ph in enumerate((p0_h, p1_h, p2_h)):
            pltpu.sync_copy(ph.at[pl.ds(base0, ppw)], pb[k])

        def norm_to_coord(t, extent):
            t = t * inv_scale
            t = t + 0.5
            t = jnp.where(t >= 1.0, jnp.float32(1 - 10e-4), t)
            t = jnp.where(t < 0.0, jnp.float32(0.0), t)
            g = 2.0 * t - 1.0
            f = (g + 1.0) * 0.5 * (extent - 1)
            f = jnp.minimum(jnp.maximum(f, 0.0), jnp.float32(extent - 1))
            return f

        def compute_and_fire(plane, ch2):
            xb = pb[plane_xy[plane][0]]
            yb = pb[plane_xy[plane][1]]
            cb = ch2 * CHUNK
            for g in range(CHUNK // LANES):
                fx = norm_to_coord(xb[pl.ds(cb + g * LANES, LANES)], W)
                fy = norm_to_coord(yb[pl.ds(cb + g * LANES, LANES)], H)
                x0 = fx.astype(jnp.int32)
                y0 = fy.astype(jnp.int32)
                wx = fx - x0.astype(jnp.float32)
                wy = fy - y0.astype(jnp.float32)
                idx = y0 * W + x0
                s = pl.ds(g * LANES, LANES)
                idxb[plane][0][s] = idx
                idxb[plane][1][s] = idx + 1
                idxb[plane][2][s] = idx + W
                idxb[plane][3][s] = idx + W + 1
                wbuf[plane][0][s] = wx
                wbuf[plane][1][s] = wy
            for k in range(4):
                pltpu.async_copy(tabs_h[plane].at[idxb[plane][k]],
                                 rows[plane].at[pl.ds(k * CHUNK, CHUNK)],
                                 gsem[plane])

        def out_slice(plane, ch, npts=CHUNK):
            base = base0 + ch * CHUNK
            return out_h.at[pl.ds(base, npts), pl.ds(plane * C, C)]

        def wait_gathers(plane):
            pltpu.make_async_copy(tabs_h[plane].at[pl.ds(0, 4 * CHUNK)],
                                  rows[plane], gsem[plane]).wait()

        def do_combine(plane, npts):
            rb = rows[plane]

            @plsc.parallel_loop(0, npts // LANES, step=1)
            def comb(g2):
                gs = pl.ds(g2 * LANES, LANES)
                wxv = wbuf[plane][0][gs]
                wyv = wbuf[plane][1][gs]
                for i2 in range(LANES):
                    i = g2 * LANES + i2
                    wx = wxv[i2]
                    wy = wyv[i2]
                    for j in range(C // LANES):
                        ls = pl.ds(j * LANES, LANES)
                        t0 = rb[i, ls]
                        t1 = rb[i + 2 * CHUNK, ls]
                        h0 = t0 + wx * (rb[i + CHUNK, ls] - t0)
                        h1 = t1 + wx * (rb[i + 3 * CHUNK, ls] - t1)
                        outv[plane][i, ls] = h0 + wy * (h1 - h0)

        def combine(plane, ch):
            wait_gathers(plane)

            @pl.when(ch > 0)
            def _wait_prev_store():
                pltpu.make_async_copy(outv[plane], out_slice(plane, ch - 1),
                                      osem[plane]).wait()

            do_combine(plane, CHUNK)
            pltpu.async_copy(outv[plane], out_slice(plane, ch), osem[plane])

        for plane in range(NPL):
            compute_and_fire(plane, 0)

        def chunk_body(ch, carry):
            for plane in range(NPL):
                combine(plane, ch)

                @pl.when(ch + 1 < nfull_w)
                def _fire_next(plane=plane):
                    compute_and_fire(plane, ch + 1)
            return carry

        lax.fori_loop(0, nfull_w, chunk_body, 0)

        if tail:
            @pl.when(has_tail)
            def _tail():
                for plane in range(NPL):
                    compute_and_fire(plane, nfull_w)
                for plane in range(NPL):
                    wait_gathers(plane)
                    pltpu.make_async_copy(
                        outv[plane], out_slice(plane, nfull_w - 1),
                        osem[plane]).wait()
                    do_combine(plane, tail)
                    pltpu.async_copy(outv[plane].at[pl.ds(0, tail)],
                                     out_slice(plane, nfull_w, tail),
                                     osem[plane])
                for plane in range(NPL):
                    pltpu.make_async_copy(outv[plane].at[pl.ds(0, tail)],
                                          out_slice(plane, nfull_w, tail),
                                          osem[plane]).wait()

            @pl.when(jnp.logical_not(has_tail))
            def _no_tail():
                for plane in range(NPL):
                    pltpu.make_async_copy(outv[plane],
                                          out_slice(plane, nfull_w - 1),
                                          osem[plane]).wait()
        else:
            for plane in range(NPL):
                pltpu.make_async_copy(outv[plane],
                                      out_slice(plane, nfull_w - 1),
                                      osem[plane]).wait()

    scratch = (
        [pltpu.VMEM((ppw,), jnp.float32) for _ in range(3)]
        + [pltpu.VMEM((CHUNK,), jnp.int32) for _ in range(4 * NPL)]
        + [pltpu.VMEM((CHUNK,), jnp.float32) for _ in range(2 * NPL)]
        + [pltpu.VMEM((4 * CHUNK, C), jnp.float32) for _ in range(NPL)]
        + [pltpu.VMEM((CHUNK, C), jnp.float32) for _ in range(NPL)]
        + [pltpu.SemaphoreType.DMA for _ in range(2 * NPL)]
    )
    return pl.kernel(
        body,
        out_type=jax.ShapeDtypeStruct((N, NPL * C), jnp.float32),
        mesh=mesh,
        scratch_types=scratch,
    ), p_pad


@jax.jit
def kernel(p, c_xz, c_xy, c_yz):
    B, N, _ = p.shape
    _, C, Hh, Ww = c_xz.shape
    tabs = [_tc_transpose(c[0].reshape(C, Hh * Ww))
            for c in (c_xz, c_xy, c_yz)]
    sampler, p_pad = _make_sc_sampler(N, Hh, Ww, C)
    pt = jnp.pad(p[0].T, ((0, 0), (0, p_pad - N)))
    out = sampler(pt[0], pt[1], pt[2], *tabs)
    return out[None]
